# baseline (speedup 1.0000x reference)
import functools

import jax
import jax.numpy as jnp
from jax import lax
from jax.experimental import pallas as pl
from jax.experimental.pallas import tpu as pltpu
from jax.experimental.pallas import tpu_sc as plsc

_NUM_CORES = 2
_NUM_SUBCORES = 16
_NUM_WORKERS = _NUM_CORES * _NUM_SUBCORES
_LANES = 16
_BB = 128
_DG = 8


@functools.partial(jax.jit, static_argnames=("bsz", "seq", "d"))
def _sc_gather(idx, table, *, bsz, seq, d):
    n_bblocks = bsz // _BB
    n_units = seq * n_bblocks
    units_per_w = n_units // _NUM_WORKERS
    b_per_w = units_per_w * _BB
    mesh = plsc.VectorSubcoreMesh(core_axis_name="c", subcore_axis_name="s")

    @functools.partial(
        pl.kernel,
        mesh=mesh,
        out_type=jax.ShapeDtypeStruct(
            (seq, d // _DG, n_bblocks, _DG, _BB), table.dtype),
        scratch_types=[
            pltpu.VMEM((b_per_w,), jnp.int32),
            pltpu.VMEM((_BB, d), table.dtype),
            pltpu.VMEM((_BB, d), table.dtype),
            pltpu.VMEM((d // _DG, _DG, _BB), table.dtype),
            pltpu.VMEM((d // _DG, _DG, _BB), table.dtype),
            pltpu.SemaphoreType.DMA,
            pltpu.SemaphoreType.DMA,
            pltpu.SemaphoreType.DMA,
            pltpu.SemaphoreType.DMA,
        ],
        compiler_params=pltpu.CompilerParams(
            use_tc_tiling_on_sc=False, needs_layout_passes=False),
    )
    def run(idx_hbm, table_hbm, out_hbm, idx_v, rows0, rows1, t0, t1,
            sem_g0, sem_g1, sem_s0, sem_s1):
        wid = lax.axis_index("s") * _NUM_CORES + lax.axis_index("c")
        base = wid * b_per_w
        pltpu.sync_copy(idx_hbm.at[pl.ds(base, b_per_w)], idx_v)

        rows = (rows0, rows1)
        tbufs = (t0, t1)
        sem_g = (sem_g0, sem_g1)
        sem_s = (sem_s0, sem_s1)
        iota16 = lax.iota(jnp.int32, _LANES)
        bb_vecs = [iota16 + 16 * q for q in range(_BB // _LANES)]

        def start_gather(i, b):
            off = pl.multiple_of(i * _BB, _BB)
            pltpu.async_copy(
                table_hbm.at[idx_v.at[pl.ds(off, _BB)]], rows[b], sem_g[b])

        def wait_gather(i, b):
            off = pl.multiple_of(i * _BB, _BB)
            pltpu.make_async_copy(
                table_hbm.at[idx_v.at[pl.ds(off, _BB)]],
                rows[b], sem_g[b]).wait()

        def out_slice(i):
            u = wid * units_per_w + i
            return out_hbm.at[u // n_bblocks, :, u % n_bblocks]

        start_gather(0, 0)

        def group(g, carry):
            for b in range(2):
                i = g * 2 + b

                @pl.when(i + 1 < units_per_w)
                def _():
                    start_gather(i + 1, 1 - b)

                wait_gather(i, b)

                @pl.when(i >= 2)
                def _():
                    pltpu.make_async_copy(
                        tbufs[b], out_hbm.at[0, :, 0], sem_s[b]).wait()

                for dd in range(d):
                    dvec = jnp.full((_LANES,), dd, jnp.int32)
                    for q in range(_BB // _LANES):
                        vec = plsc.load_gather(rows[b], [bb_vecs[q], dvec])
                        tbufs[b][dd // _DG, dd % _DG, pl.ds(16 * q, 16)] = vec

                pltpu.async_copy(tbufs[b], out_slice(i), sem_s[b])
            return carry

        lax.fori_loop(0, units_per_w // 2, group, 0)
        for b in range(2):
            pltpu.make_async_copy(
                tbufs[b], out_hbm.at[0, :, 0], sem_s[b]).wait()

    return run(idx, table)


def kernel(token_ids, embedding_matrix):
    bsz, seq = token_ids.shape
    b = bsz * seq
    v, d = embedding_matrix.shape
    idx_t = token_ids.T.reshape(b).astype(jnp.int32) * 2
    table128 = jnp.pad(embedding_matrix, ((0, 0), (0, 128 - d)))
    table_lin = table128.reshape(2 * v, d)
    out5 = _sc_gather(idx_t, table_lin, bsz=bsz, seq=seq, d=d)
    return out5.transpose(2, 4, 0, 1, 3).reshape(bsz, seq, d)


# speedup vs baseline: 1.8662x; 1.8662x over previous
import functools

import jax
import jax.numpy as jnp
from jax import lax
from jax.experimental import pallas as pl
from jax.experimental.pallas import tpu as pltpu
from jax.experimental.pallas import tpu_sc as plsc

_NUM_CORES = 2
_NUM_SUBCORES = 16
_NUM_WORKERS = _NUM_CORES * _NUM_SUBCORES
_LANES = 16
_BB = 128
_DG = 8
_TPAD = 133


@functools.partial(jax.jit, static_argnames=("bsz", "seq", "d"))
def _sc_gather(idx, table, *, bsz, seq, d):
    n_bblocks = bsz // _BB
    n_units = seq * n_bblocks
    units_per_w = n_units // _NUM_WORKERS
    b_per_w = units_per_w * _BB
    mesh = plsc.VectorSubcoreMesh(core_axis_name="c", subcore_axis_name="s")

    @functools.partial(
        pl.kernel,
        mesh=mesh,
        out_type=jax.ShapeDtypeStruct(
            (seq, d // _DG, n_bblocks, _DG, _BB), table.dtype),
        scratch_types=[
            pltpu.VMEM((b_per_w,), jnp.int32),
            pltpu.VMEM((_BB, d), table.dtype),
            pltpu.VMEM((_BB, d), table.dtype),
            pltpu.VMEM((d // _DG, _DG, _TPAD), table.dtype),
            pltpu.VMEM((d // _DG, _DG, _TPAD), table.dtype),
            pltpu.SemaphoreType.DMA,
            pltpu.SemaphoreType.DMA,
            pltpu.SemaphoreType.DMA,
            pltpu.SemaphoreType.DMA,
        ],
        compiler_params=pltpu.CompilerParams(
            use_tc_tiling_on_sc=False, needs_layout_passes=False),
    )
    def run(idx_hbm, table_hbm, out_hbm, idx_v, rows0, rows1, t0, t1,
            sem_g0, sem_g1, sem_s0, sem_s1):
        wid = lax.axis_index("s") * _NUM_CORES + lax.axis_index("c")
        base = wid * b_per_w
        pltpu.sync_copy(idx_hbm.at[pl.ds(base, b_per_w)], idx_v)

        rows = (rows0, rows1)
        tbufs = (t0, t1)
        sem_g = (sem_g0, sem_g1)
        sem_s = (sem_s0, sem_s1)
        iota16 = lax.iota(jnp.int32, _LANES)
        nq = d // _LANES
        dg_vecs = [(16 * q + iota16) // _DG for q in range(nq)]
        dd_vecs = [(16 * q + iota16) % _DG for q in range(nq)]

        def start_gather(i, b):
            off = pl.multiple_of(i * _BB, _BB)
            pltpu.async_copy(
                table_hbm.at[idx_v.at[pl.ds(off, _BB)]], rows[b], sem_g[b])

        def wait_gather(i, b):
            off = pl.multiple_of(i * _BB, _BB)
            pltpu.make_async_copy(
                table_hbm.at[idx_v.at[pl.ds(off, _BB)]],
                rows[b], sem_g[b]).wait()

        def out_slice(i):
            u = wid * units_per_w + i
            return out_hbm.at[u // n_bblocks, :, u % n_bblocks]

        start_gather(0, 0)

        def group(g, carry):
            for b in range(2):
                i = g * 2 + b

                @pl.when(i + 1 < units_per_w)
                def _():
                    start_gather(i + 1, 1 - b)

                wait_gather(i, b)

                @pl.when(i >= 2)
                def _():
                    pltpu.make_async_copy(
                        tbufs[b].at[:, :, pl.ds(0, _BB)],
                        out_hbm.at[0, :, 0], sem_s[b]).wait()

                for bb in range(_BB):
                    bb_vec = jnp.full((_LANES,), bb, jnp.int32)
                    for q in range(nq):
                        vec = rows[b][bb, pl.ds(16 * q, 16)]
                        plsc.store_scatter(
                            tbufs[b], [dg_vecs[q], dd_vecs[q], bb_vec], vec)

                pltpu.async_copy(
                    tbufs[b].at[:, :, pl.ds(0, _BB)], out_slice(i), sem_s[b])
            return carry

        lax.fori_loop(0, units_per_w // 2, group, 0)
        for b in range(2):
            pltpu.make_async_copy(
                tbufs[b].at[:, :, pl.ds(0, _BB)],
                out_hbm.at[0, :, 0], sem_s[b]).wait()

    return run(idx, table)


def kernel(token_ids, embedding_matrix):
    bsz, seq = token_ids.shape
    b = bsz * seq
    v, d = embedding_matrix.shape
    idx_t = token_ids.T.reshape(b).astype(jnp.int32) * 2
    table128 = jnp.pad(embedding_matrix, ((0, 0), (0, 128 - d)))
    table_lin = table128.reshape(2 * v, d)
    out5 = _sc_gather(idx_t, table_lin, bsz=bsz, seq=seq, d=d)
    return out5.transpose(2, 4, 0, 1, 3).reshape(bsz, seq, d)


# speedup vs baseline: 2.1121x; 1.1318x over previous
import functools

import jax
import jax.numpy as jnp
from jax import lax
from jax.experimental import pallas as pl
from jax.experimental.pallas import tpu as pltpu
from jax.experimental.pallas import tpu_sc as plsc

_NUM_CORES = 2
_NUM_SUBCORES = 16
_NUM_WORKERS = _NUM_CORES * _NUM_SUBCORES
_LANES = 16
_BB = 128
_DG = 8
_TPAD = 133


@functools.partial(jax.jit, static_argnames=("bsz", "seq", "d"))
def _sc_gather(idx, table, *, bsz, seq, d):
    n_bblocks = bsz // _BB
    n_units = seq * n_bblocks
    units_per_w = n_units // _NUM_WORKERS
    b_per_w = units_per_w * _BB
    mesh = plsc.VectorSubcoreMesh(core_axis_name="c", subcore_axis_name="s")

    @functools.partial(
        pl.kernel,
        mesh=mesh,
        out_type=jax.ShapeDtypeStruct(
            (seq, d // _DG, n_bblocks, _DG, _BB), table.dtype),
        scratch_types=[
            pltpu.VMEM((b_per_w,), jnp.int32),
            pltpu.VMEM((_BB, d), table.dtype),
            pltpu.VMEM((_BB, d), table.dtype),
            pltpu.VMEM((d // _DG, _DG, _TPAD), table.dtype),
            pltpu.VMEM((d // _DG, _DG, _TPAD), table.dtype),
            pltpu.SemaphoreType.DMA,
            pltpu.SemaphoreType.DMA,
            pltpu.SemaphoreType.DMA,
            pltpu.SemaphoreType.DMA,
        ],
        compiler_params=pltpu.CompilerParams(
            use_tc_tiling_on_sc=False, needs_layout_passes=False),
    )
    def run(idx_hbm, table_hbm, out_hbm, idx_v, rows0, rows1, t0, t1,
            sem_g0, sem_g1, sem_s0, sem_s1):
        wid = lax.axis_index("s") * _NUM_CORES + lax.axis_index("c")
        base = wid * b_per_w
        pltpu.sync_copy(idx_hbm.at[pl.ds(base, b_per_w)], idx_v)

        rows = (rows0, rows1)
        tbufs = (t0, t1)
        sem_g = (sem_g0, sem_g1)
        sem_s = (sem_s0, sem_s1)
        iota16 = lax.iota(jnp.int32, _LANES)
        nq = d // _LANES
        dg_vecs = [(16 * q + iota16) // _DG for q in range(nq)]
        dd_vecs = [(16 * q + iota16) % _DG for q in range(nq)]

        def start_gather(i, b):
            off = pl.multiple_of(i * _BB, _BB)
            pltpu.async_copy(
                table_hbm.at[idx_v.at[pl.ds(off, _BB)]], rows[b], sem_g[b])

        def wait_gather(i, b):
            off = pl.multiple_of(i * _BB, _BB)
            pltpu.make_async_copy(
                table_hbm.at[idx_v.at[pl.ds(off, _BB)]],
                rows[b], sem_g[b]).wait()

        def out_slice(i):
            u = wid * units_per_w + i
            return out_hbm.at[u // n_bblocks, :, u % n_bblocks]

        start_gather(0, 0)

        def group(g, carry):
            for b in range(2):
                i = g * 2 + b

                @pl.when(i + 1 < units_per_w)
                def _():
                    start_gather(i + 1, 1 - b)

                wait_gather(i, b)

                @pl.when(i >= 2)
                def _():
                    pltpu.make_async_copy(
                        tbufs[b].at[:, :, pl.ds(0, _BB)],
                        out_hbm.at[0, :, 0], sem_s[b]).wait()

                def bbg_body(bbg, c, _b=b):
                    for j in range(8):
                        bbi = bbg * 8 + j
                        bb_vec = bbi + jnp.zeros((_LANES,), jnp.int32)
                        for q in range(nq):
                            vec = rows[_b][bbi, pl.ds(16 * q, 16)]
                            plsc.store_scatter(
                                tbufs[_b],
                                [dg_vecs[q], dd_vecs[q], bb_vec], vec)
                    return c

                lax.fori_loop(0, _BB // 8, bbg_body, 0)

                pltpu.async_copy(
                    tbufs[b].at[:, :, pl.ds(0, _BB)], out_slice(i), sem_s[b])
            return carry

        lax.fori_loop(0, units_per_w // 2, group, 0)
        for b in range(2):
            pltpu.make_async_copy(
                tbufs[b].at[:, :, pl.ds(0, _BB)],
                out_hbm.at[0, :, 0], sem_s[b]).wait()

    return run(idx, table)


def kernel(token_ids, embedding_matrix):
    bsz, seq = token_ids.shape
    b = bsz * seq
    v, d = embedding_matrix.shape
    idx_t = token_ids.T.reshape(b).astype(jnp.int32) * 2
    table128 = jnp.pad(embedding_matrix, ((0, 0), (0, 128 - d)))
    table_lin = table128.reshape(2 * v, d)
    out5 = _sc_gather(idx_t, table_lin, bsz=bsz, seq=seq, d=d)
    return out5.transpose(2, 4, 0, 1, 3).reshape(bsz, seq, d)


# speedup vs baseline: 2.1123x; 1.0001x over previous
import functools

import jax
import jax.numpy as jnp
from jax import lax
from jax.experimental import pallas as pl
from jax.experimental.pallas import tpu as pltpu
from jax.experimental.pallas import tpu_sc as plsc

_NUM_CORES = 2
_NUM_SUBCORES = 16
_NUM_WORKERS = _NUM_CORES * _NUM_SUBCORES
_LANES = 16
_BB = 128
_DG = 8
_TPAD = 133


@functools.partial(jax.jit, static_argnames=("bsz", "seq", "d"))
def _sc_gather(idx, table, *, bsz, seq, d):
    n_bblocks = bsz // _BB
    n_units = seq * n_bblocks
    units_per_w = n_units // _NUM_WORKERS
    b_per_w = units_per_w * _BB
    mesh = plsc.VectorSubcoreMesh(core_axis_name="c", subcore_axis_name="s")

    @functools.partial(
        pl.kernel,
        mesh=mesh,
        out_type=jax.ShapeDtypeStruct(
            (seq, d // _DG, n_bblocks, _DG, _BB), table.dtype),
        scratch_types=[
            pltpu.VMEM((b_per_w,), jnp.int32),
            pltpu.VMEM((_BB, d), table.dtype),
            pltpu.VMEM((_BB, d), table.dtype),
            pltpu.VMEM((d // _DG, _DG, _TPAD), table.dtype),
            pltpu.VMEM((d // _DG, _DG, _TPAD), table.dtype),
            pltpu.SemaphoreType.DMA,
            pltpu.SemaphoreType.DMA,
            pltpu.SemaphoreType.DMA,
            pltpu.SemaphoreType.DMA,
        ],
        compiler_params=pltpu.CompilerParams(
            use_tc_tiling_on_sc=False, needs_layout_passes=False),
    )
    def run(idx_hbm, table_hbm, out_hbm, idx_v, rows0, rows1, t0, t1,
            sem_g0, sem_g1, sem_s0, sem_s1):
        wid = lax.axis_index("s") * _NUM_CORES + lax.axis_index("c")
        base = wid * b_per_w
        pltpu.sync_copy(idx_hbm.at[pl.ds(base, b_per_w)], idx_v)

        rows = (rows0, rows1)
        tbufs = (t0, t1)
        sem_g = (sem_g0, sem_g1)
        sem_s = (sem_s0, sem_s1)
        iota16 = lax.iota(jnp.int32, _LANES)
        nq = d // _LANES
        dg_vecs = [(16 * q + iota16) // _DG for q in range(nq)]
        dd_vecs = [(16 * q + iota16) % _DG for q in range(nq)]

        def start_gather(i, b):
            off = pl.multiple_of(i * _BB, _BB)
            pltpu.async_copy(
                table_hbm.at[idx_v.at[pl.ds(off, _BB)]], rows[b], sem_g[b])

        def wait_gather(i, b):
            off = pl.multiple_of(i * _BB, _BB)
            pltpu.make_async_copy(
                table_hbm.at[idx_v.at[pl.ds(off, _BB)]],
                rows[b], sem_g[b]).wait()

        def out_slice(i):
            u = wid * units_per_w + i
            return out_hbm.at[u // n_bblocks, :, u % n_bblocks]

        start_gather(0, 0)

        def group(g, carry):
            for b in range(2):
                i = g * 2 + b

                @pl.when(i + 1 < units_per_w)
                def _():
                    start_gather(i + 1, 1 - b)

                wait_gather(i, b)

                @pl.when(i >= 2)
                def _():
                    pltpu.make_async_copy(
                        tbufs[b].at[:, :, pl.ds(0, _BB)],
                        out_hbm.at[0, :, 0], sem_s[b]).wait()

                def bbg_body(bbg, c, _b=b):
                    for j in range(16):
                        bbi = bbg * 16 + j
                        bb_vec = bbi + jnp.zeros((_LANES,), jnp.int32)
                        for q in range(nq):
                            vec = rows[_b][bbi, pl.ds(16 * q, 16)]
                            plsc.store_scatter(
                                tbufs[_b],
                                [dg_vecs[q], dd_vecs[q], bb_vec], vec)
                    return c

                lax.fori_loop(0, _BB // 16, bbg_body, 0)

                pltpu.async_copy(
                    tbufs[b].at[:, :, pl.ds(0, _BB)], out_slice(i), sem_s[b])
            return carry

        lax.fori_loop(0, units_per_w // 2, group, 0)
        for b in range(2):
            pltpu.make_async_copy(
                tbufs[b].at[:, :, pl.ds(0, _BB)],
                out_hbm.at[0, :, 0], sem_s[b]).wait()

    return run(idx, table)


def kernel(token_ids, embedding_matrix):
    bsz, seq = token_ids.shape
    b = bsz * seq
    v, d = embedding_matrix.shape
    idx_t = token_ids.T.reshape(b).astype(jnp.int32) * 2
    table128 = jnp.pad(embedding_matrix, ((0, 0), (0, 128 - d)))
    table_lin = table128.reshape(2 * v, d)
    out5 = _sc_gather(idx_t, table_lin, bsz=bsz, seq=seq, d=d)
    return out5.transpose(2, 4, 0, 1, 3).reshape(bsz, seq, d)
